# baseline (speedup 1.0000x reference)
import functools

import jax
import jax.numpy as jnp
from jax import lax
from jax.experimental import pallas as pl
from jax.experimental.pallas import tpu as pltpu
from jax.experimental.pallas import tpu_sc as plsc

_NC = 2
_NS = 16
_NW = _NC * _NS
_CHUNK = 128


@functools.partial(jax.jit, static_argnames=("b_total", "d"))
def _sc_gather(index_flat, emb_weight, *, b_total, d):
    b_per_w = b_total // _NW
    n_chunks = b_per_w // _CHUNK
    mesh = plsc.VectorSubcoreMesh(core_axis_name="c", subcore_axis_name="s")

    @functools.partial(
        pl.kernel,
        out_type=jax.ShapeDtypeStruct((b_total, d), jnp.float32),
        mesh=mesh,
        scratch_types=[
            pltpu.VMEM((b_per_w,), jnp.int32),
            pltpu.VMEM((_CHUNK, d), jnp.float32),
            pltpu.SemaphoreType.DMA,
        ],
        compiler_params=pltpu.CompilerParams(use_tc_tiling_on_sc=False),
    )
    def gather_kernel(idx_hbm, table_hbm, out_hbm, idx_v, rows_v, sem):
        wid = lax.axis_index("s") * _NC + lax.axis_index("c")
        base = wid * b_per_w
        pltpu.sync_copy(idx_hbm.at[pl.ds(base, b_per_w)], idx_v)

        def body(i, carry):
            off = pl.multiple_of(i * _CHUNK, _CHUNK)
            pltpu.async_copy(
                table_hbm.at[idx_v.at[pl.ds(off, _CHUNK)]], rows_v, sem
            ).wait()
            pltpu.sync_copy(rows_v, out_hbm.at[pl.ds(base + off, _CHUNK)])
            return carry

        lax.fori_loop(0, n_chunks, body, 0)

    return gather_kernel(index_flat, emb_weight)


def kernel(index, emb_weight):
    b, s = index.shape
    d = emb_weight.shape[1]
    out = _sc_gather(index.reshape(-1), emb_weight, b_total=b * s, d=d)
    return out.reshape(b, s, d)


# speedup vs baseline: 1.0449x; 1.0449x over previous
import functools

import jax
import jax.numpy as jnp
from jax import lax
from jax.experimental import pallas as pl
from jax.experimental.pallas import tpu as pltpu
from jax.experimental.pallas import tpu_sc as plsc

_NC = 2
_NS = 16
_NW = _NC * _NS
_CHUNK = 128
_NBUF = 5


@functools.partial(jax.jit, static_argnames=("b_total", "d"))
def _sc_gather(index_flat, emb_weight, *, b_total, d):
    b_per_w = b_total // _NW
    n_chunks = b_per_w // _CHUNK
    mesh = plsc.VectorSubcoreMesh(core_axis_name="c", subcore_axis_name="s")

    @functools.partial(
        pl.kernel,
        out_type=jax.ShapeDtypeStruct((b_total, d), jnp.float32),
        mesh=mesh,
        scratch_types=[
            pltpu.VMEM((b_per_w,), jnp.int32),
            pltpu.VMEM((_NBUF, _CHUNK, d), jnp.float32),
        ] + [pltpu.SemaphoreType.DMA] * _NBUF,
        compiler_params=pltpu.CompilerParams(use_tc_tiling_on_sc=False),
    )
    def gather_kernel(idx_hbm, table_hbm, out_hbm, idx_v, rows_v, *sems):
        wid = lax.axis_index("s") * _NC + lax.axis_index("c")
        base = wid * b_per_w
        pltpu.sync_copy(idx_hbm.at[pl.ds(base, b_per_w)], idx_v)

        def fire(i, b):
            off = pl.multiple_of(i * _CHUNK, _CHUNK)
            pltpu.async_copy(
                table_hbm.at[idx_v.at[pl.ds(off, _CHUNK)]],
                rows_v.at[b],
                sems[b],
            )

        for b in range(_NBUF):
            fire(b, b)

        def outer(g, carry):
            for b in range(_NBUF):
                i = g * _NBUF + b
                off = pl.multiple_of(i * _CHUNK, _CHUNK)
                pltpu.make_async_copy(
                    table_hbm.at[idx_v.at[pl.ds(off, _CHUNK)]],
                    rows_v.at[b],
                    sems[b],
                ).wait()
                pltpu.sync_copy(rows_v.at[b], out_hbm.at[pl.ds(base + off, _CHUNK)])
                nxt = i + _NBUF

                @pl.when(nxt < n_chunks)
                def _():
                    fire(nxt, b)

            return carry

        lax.fori_loop(0, n_chunks // _NBUF, outer, 0)

    return gather_kernel(index_flat, emb_weight)


def kernel(index, emb_weight):
    b, s = index.shape
    d = emb_weight.shape[1]
    out = _sc_gather(index.reshape(-1), emb_weight, b_total=b * s, d=d)
    return out.reshape(b, s, d)
